# Initial kernel scaffold; baseline (speedup 1.0000x reference)
import jax
import jax.numpy as jnp
from jax.experimental import pallas as pl


def kernel(x, W1, W2, W3, W4, g1, b1, g2, b2, g3, b3, g4, b4):
    raise NotImplementedError("write your pallas kernel here")


# speedup vs baseline: 4.3504x; 4.3504x over previous
import functools

import jax
import jax.numpy as jnp
from jax.experimental import pallas as pl
from jax.experimental.pallas import tpu as pltpu

_K = 20
_NEG = -3.0e38


def _layer_a_body(xt_ref, wT_ref,
                  ymax_ref, ymin_ref, s1_ref, s2_ref,
                  d_ref, ymaxs_ref, ymins_ref, ysum_ref, ysq_ref):
    xt = xt_ref[0]
    n = xt.shape[0]
    f32 = jnp.float32
    hi = jax.lax.Precision.HIGHEST
    wT = wT_ref[...]
    x2 = xt * xt
    xx_col = jnp.sum(x2, axis=1, keepdims=True)
    r_i = jax.lax.broadcasted_iota(jnp.int32, (n, n), 0)
    c_i = jax.lax.broadcasted_iota(jnp.int32, (n, n), 1)
    eye = jnp.where(r_i == c_i, 1.0, 0.0).astype(f32)
    xx_row = jax.lax.dot_general(xx_col, eye, (((0,), (0,)), ((), ())),
                                 precision=hi, preferred_element_type=f32)
    inner = jax.lax.dot_general(xt, xt, (((1,), (1,)), ((), ())),
                                preferred_element_type=f32)
    d_ref[...] = 2.0 * inner - xx_col - xx_row

    ymaxs_ref[...] = jnp.full(ymaxs_ref.shape, _NEG, f32)
    ymins_ref[...] = jnp.full(ymins_ref.shape, -_NEG, f32)
    ysum_ref[...] = jnp.zeros(ysum_ref.shape, f32)
    ysq_ref[...] = jnp.zeros(ysq_ref.shape, f32)

    def body(_, carry):
        d = d_ref[...]
        m = jnp.max(d, axis=1, keepdims=True)
        masked = jnp.where(d == m, c_i, n)
        amin = jnp.min(masked, axis=1, keepdims=True)
        sel = masked == amin
        onehot = jnp.where(sel, 1.0, 0.0).astype(f32)
        xg = jax.lax.dot_general(onehot, xt, (((1,), (0,)), ((), ())),
                                 precision=hi, preferred_element_type=f32)
        e = jnp.concatenate([xg - xt, xt], axis=1)
        y = jax.lax.dot_general(e, wT, (((1,), (0,)), ((), ())),
                                preferred_element_type=f32)
        d_ref[...] = jnp.where(sel, _NEG, d)
        ymaxs_ref[...] = jnp.maximum(ymaxs_ref[...], y)
        ymins_ref[...] = jnp.minimum(ymins_ref[...], y)
        ysum_ref[...] = ysum_ref[...] + y
        ysq_ref[...] = ysq_ref[...] + y * y
        return carry

    jax.lax.fori_loop(0, _K, body, 0)

    ymax_ref[0] = ymaxs_ref[...]
    ymin_ref[0] = ymins_ref[...]
    s1_ref[0] = jnp.sum(ysum_ref[...], axis=0, keepdims=True)
    s2_ref[0] = jnp.sum(ysq_ref[...], axis=0, keepdims=True)


def _layer_b_body(ymax_ref, ymin_ref, s1_ref, s2_ref, g_ref, b_ref, out_ref,
                  *, count):
    s1 = jnp.sum(s1_ref[...], axis=0)
    s2 = jnp.sum(s2_ref[...], axis=0)
    mean = s1 / count
    var = s2 / count - mean * mean
    scale = g_ref[...] / jnp.sqrt(var + 1e-5)
    shift = b_ref[...] - mean * scale
    ysel = jnp.where(scale >= 0.0, ymax_ref[0], ymin_ref[0])
    t = ysel * scale + shift
    out_ref[0] = jnp.where(t >= 0.0, t, 0.2 * t)


def _edge_layer(xt, W, g, b):
    B, N, C = xt.shape
    O = W.shape[0]
    f32 = jnp.float32
    wT = jnp.transpose(W)

    ymax, ymin, s1, s2 = pl.pallas_call(
        _layer_a_body,
        grid=(B,),
        in_specs=[
            pl.BlockSpec((1, N, C), lambda i: (i, 0, 0)),
            pl.BlockSpec((2 * C, O), lambda i: (0, 0)),
        ],
        out_specs=[
            pl.BlockSpec((1, N, O), lambda i: (i, 0, 0)),
            pl.BlockSpec((1, N, O), lambda i: (i, 0, 0)),
            pl.BlockSpec((1, 1, O), lambda i: (i, 0, 0)),
            pl.BlockSpec((1, 1, O), lambda i: (i, 0, 0)),
        ],
        out_shape=[
            jax.ShapeDtypeStruct((B, N, O), f32),
            jax.ShapeDtypeStruct((B, N, O), f32),
            jax.ShapeDtypeStruct((B, 1, O), f32),
            jax.ShapeDtypeStruct((B, 1, O), f32),
        ],
        scratch_shapes=[
            pltpu.VMEM((N, N), f32),
            pltpu.VMEM((N, O), f32),
            pltpu.VMEM((N, O), f32),
            pltpu.VMEM((N, O), f32),
            pltpu.VMEM((N, O), f32),
        ],
    )(xt, wT)

    out = pl.pallas_call(
        functools.partial(_layer_b_body, count=float(B * N * _K)),
        grid=(B,),
        in_specs=[
            pl.BlockSpec((1, N, O), lambda i: (i, 0, 0)),
            pl.BlockSpec((1, N, O), lambda i: (i, 0, 0)),
            pl.BlockSpec((B, 1, O), lambda i: (0, 0, 0)),
            pl.BlockSpec((B, 1, O), lambda i: (0, 0, 0)),
            pl.BlockSpec((1, O), lambda i: (0, 0)),
            pl.BlockSpec((1, O), lambda i: (0, 0)),
        ],
        out_specs=pl.BlockSpec((1, N, O), lambda i: (i, 0, 0)),
        out_shape=jax.ShapeDtypeStruct((B, N, O), f32),
    )(ymax, ymin, s1, s2, g.reshape(1, O), b.reshape(1, O))
    return out


def kernel(x, W1, W2, W3, W4, g1, b1, g2, b2, g3, b3, g4, b4):
    h = _edge_layer(x, W1, g1, b1)
    h = _edge_layer(h, W2, g2, b2)
    h = _edge_layer(h, W3, g3, b3)
    h = _edge_layer(h, W4, g4, b4)
    return h


# speedup vs baseline: 4.3522x; 1.0004x over previous
import functools

import jax
import jax.numpy as jnp
from jax.experimental import pallas as pl
from jax.experimental.pallas import tpu as pltpu

_K = 20
_NEG = -3.0e38


def _layer_a_body(xt_ref, wT_ref,
                  ymax_ref, ymin_ref, s1_ref, s2_ref,
                  d_ref, ymaxs_ref, ymins_ref, ysum_ref, ysq_ref):
    xt = xt_ref[0]
    n = xt.shape[0]
    f32 = jnp.float32
    hi = jax.lax.Precision.HIGHEST
    wT = wT_ref[...]
    x2 = xt * xt
    xx_col = jnp.sum(x2, axis=1, keepdims=True)
    r_i = jax.lax.broadcasted_iota(jnp.int32, (n, n), 0)
    c_i = jax.lax.broadcasted_iota(jnp.int32, (n, n), 1)
    eye = jnp.where(r_i == c_i, 1.0, 0.0).astype(f32)
    xx_row = jax.lax.dot_general(xx_col, eye, (((0,), (0,)), ((), ())),
                                 precision=hi, preferred_element_type=f32)
    inner = jax.lax.dot_general(xt, xt, (((1,), (1,)), ((), ())),
                                preferred_element_type=f32)
    d_ref[...] = 2.0 * inner - xx_col - xx_row

    ymaxs_ref[...] = jnp.full(ymaxs_ref.shape, _NEG, f32)
    ymins_ref[...] = jnp.full(ymins_ref.shape, -_NEG, f32)
    ysum_ref[...] = jnp.zeros(ysum_ref.shape, f32)
    ysq_ref[...] = jnp.zeros(ysq_ref.shape, f32)

    def body(_, carry):
        d = d_ref[...]
        m = jnp.max(d, axis=1, keepdims=True)
        masked = jnp.where(d == m, c_i, n)
        amin = jnp.min(masked, axis=1, keepdims=True)
        sel = masked == amin
        onehot = jnp.where(sel, 1.0, 0.0).astype(f32)
        xg = jax.lax.dot_general(onehot, xt, (((1,), (0,)), ((), ())),
                                 precision=hi, preferred_element_type=f32)
        e = jnp.concatenate([xg - xt, xt], axis=1)
        y = jax.lax.dot_general(e, wT, (((1,), (0,)), ((), ())),
                                preferred_element_type=f32)
        d_ref[...] = jnp.where(sel, _NEG, d)
        ymaxs_ref[...] = jnp.maximum(ymaxs_ref[...], y)
        ymins_ref[...] = jnp.minimum(ymins_ref[...], y)
        ysum_ref[...] = ysum_ref[...] + y
        ysq_ref[...] = ysq_ref[...] + y * y
        return carry

    jax.lax.fori_loop(0, _K, body, 0)

    ymax_ref[0] = ymaxs_ref[...]
    ymin_ref[0] = ymins_ref[...]
    s1_ref[0] = jnp.sum(ysum_ref[...], axis=0, keepdims=True)
    s2_ref[0] = jnp.sum(ysq_ref[...], axis=0, keepdims=True)


def _layer_b_body(ymax_ref, ymin_ref, s1_ref, s2_ref, g_ref, b_ref, out_ref,
                  *, count):
    s1 = jnp.sum(s1_ref[...], axis=0)
    s2 = jnp.sum(s2_ref[...], axis=0)
    mean = s1 / count
    var = s2 / count - mean * mean
    g = g_ref[...]
    ysel = jnp.where(g >= 0.0, ymax_ref[0], ymin_ref[0])
    t = (ysel - mean) / jnp.sqrt(var + 1e-5)
    t = t * g + b_ref[...]
    out_ref[0] = jnp.where(t >= 0.0, t, 0.2 * t)


def _edge_layer(xt, W, g, b):
    B, N, C = xt.shape
    O = W.shape[0]
    f32 = jnp.float32
    wT = jnp.transpose(W)

    ymax, ymin, s1, s2 = pl.pallas_call(
        _layer_a_body,
        grid=(B,),
        in_specs=[
            pl.BlockSpec((1, N, C), lambda i: (i, 0, 0)),
            pl.BlockSpec((2 * C, O), lambda i: (0, 0)),
        ],
        out_specs=[
            pl.BlockSpec((1, N, O), lambda i: (i, 0, 0)),
            pl.BlockSpec((1, N, O), lambda i: (i, 0, 0)),
            pl.BlockSpec((1, 1, O), lambda i: (i, 0, 0)),
            pl.BlockSpec((1, 1, O), lambda i: (i, 0, 0)),
        ],
        out_shape=[
            jax.ShapeDtypeStruct((B, N, O), f32),
            jax.ShapeDtypeStruct((B, N, O), f32),
            jax.ShapeDtypeStruct((B, 1, O), f32),
            jax.ShapeDtypeStruct((B, 1, O), f32),
        ],
        scratch_shapes=[
            pltpu.VMEM((N, N), f32),
            pltpu.VMEM((N, O), f32),
            pltpu.VMEM((N, O), f32),
            pltpu.VMEM((N, O), f32),
            pltpu.VMEM((N, O), f32),
        ],
    )(xt, wT)

    out = pl.pallas_call(
        functools.partial(_layer_b_body, count=float(B * N * _K)),
        grid=(B,),
        in_specs=[
            pl.BlockSpec((1, N, O), lambda i: (i, 0, 0)),
            pl.BlockSpec((1, N, O), lambda i: (i, 0, 0)),
            pl.BlockSpec((B, 1, O), lambda i: (0, 0, 0)),
            pl.BlockSpec((B, 1, O), lambda i: (0, 0, 0)),
            pl.BlockSpec((1, O), lambda i: (0, 0)),
            pl.BlockSpec((1, O), lambda i: (0, 0)),
        ],
        out_specs=pl.BlockSpec((1, N, O), lambda i: (i, 0, 0)),
        out_shape=jax.ShapeDtypeStruct((B, N, O), f32),
    )(ymax, ymin, s1, s2, g.reshape(1, O), b.reshape(1, O))
    return out


def kernel(x, W1, W2, W3, W4, g1, b1, g2, b2, g3, b3, g4, b4):
    h = _edge_layer(x, W1, g1, b1)
    h = _edge_layer(h, W2, g2, b2)
    h = _edge_layer(h, W3, g3, b3)
    h = _edge_layer(h, W4, g4, b4)
    return h


# speedup vs baseline: 6.0518x; 1.3905x over previous
import functools

import jax
import jax.numpy as jnp
from jax import lax
from jax.experimental import pallas as pl
from jax.experimental.pallas import tpu as pltpu
from jax.experimental.pallas import tpu_sc as plsc

_K = 20
_NEG = -3.0e38


def _pairwise_dist(xt, n):
    f32 = jnp.float32
    hi = jax.lax.Precision.HIGHEST
    x2 = xt * xt
    xx_col = jnp.sum(x2, axis=1, keepdims=True)
    r_i = jax.lax.broadcasted_iota(jnp.int32, (n, n), 0)
    c_i = jax.lax.broadcasted_iota(jnp.int32, (n, n), 1)
    eye = jnp.where(r_i == c_i, 1.0, 0.0).astype(f32)
    xx_row = jax.lax.dot_general(xx_col, eye, (((0,), (0,)), ((), ())),
                                 precision=hi, preferred_element_type=f32)
    inner = jax.lax.dot_general(xt, xt, (((1,), (1,)), ((), ())),
                                preferred_element_type=f32)
    return 2.0 * inner - xx_col - xx_row, c_i


def _layer_a_body(xt_ref, wT_ref,
                  ymax_ref, ymin_ref, s1_ref, s2_ref,
                  d_ref, ymaxs_ref, ymins_ref, ysum_ref, ysq_ref):
    xt = xt_ref[0]
    n = xt.shape[0]
    f32 = jnp.float32
    hi = jax.lax.Precision.HIGHEST
    wT = wT_ref[...]
    d, c_i = _pairwise_dist(xt, n)
    d_ref[...] = d

    ymaxs_ref[...] = jnp.full(ymaxs_ref.shape, _NEG, f32)
    ymins_ref[...] = jnp.full(ymins_ref.shape, -_NEG, f32)
    ysum_ref[...] = jnp.zeros(ysum_ref.shape, f32)
    ysq_ref[...] = jnp.zeros(ysq_ref.shape, f32)

    def body(_, carry):
        d = d_ref[...]
        m = jnp.max(d, axis=1, keepdims=True)
        masked = jnp.where(d == m, c_i, n)
        amin = jnp.min(masked, axis=1, keepdims=True)
        sel = masked == amin
        onehot = jnp.where(sel, 1.0, 0.0).astype(f32)
        xg = jax.lax.dot_general(onehot, xt, (((1,), (0,)), ((), ())),
                                 precision=hi, preferred_element_type=f32)
        e = jnp.concatenate([xg - xt, xt], axis=1)
        y = jax.lax.dot_general(e, wT, (((1,), (0,)), ((), ())),
                                preferred_element_type=f32)
        d_ref[...] = jnp.where(sel, _NEG, d)
        ymaxs_ref[...] = jnp.maximum(ymaxs_ref[...], y)
        ymins_ref[...] = jnp.minimum(ymins_ref[...], y)
        ysum_ref[...] = ysum_ref[...] + y
        ysq_ref[...] = ysq_ref[...] + y * y
        return carry

    jax.lax.fori_loop(0, _K, body, 0)

    ymax_ref[0] = ymaxs_ref[...]
    ymin_ref[0] = ymins_ref[...]
    s1_ref[0] = jnp.sum(ysum_ref[...], axis=0, keepdims=True)
    s2_ref[0] = jnp.sum(ysq_ref[...], axis=0, keepdims=True)


def _layer_b_body(ymax_ref, ymin_ref, s1_ref, s2_ref, g_ref, b_ref, out_ref,
                  *, count):
    s1 = jnp.sum(s1_ref[...], axis=0)
    s2 = jnp.sum(s2_ref[...], axis=0)
    mean = s1 / count
    var = s2 / count - mean * mean
    g = g_ref[...]
    ysel = jnp.where(g >= 0.0, ymax_ref[0], ymin_ref[0])
    t = (ysel - mean) / jnp.sqrt(var + 1e-5)
    t = t * g + b_ref[...]
    out_ref[0] = jnp.where(t >= 0.0, t, 0.2 * t)


def _bn_finalize(ymax, ymin, s1, s2, g, b):
    B, N, O = ymax.shape
    f32 = jnp.float32
    return pl.pallas_call(
        functools.partial(_layer_b_body, count=float(B * N * _K)),
        grid=(B,),
        in_specs=[
            pl.BlockSpec((1, N, O), lambda i: (i, 0, 0)),
            pl.BlockSpec((1, N, O), lambda i: (i, 0, 0)),
            pl.BlockSpec((B, 1, O), lambda i: (0, 0, 0)),
            pl.BlockSpec((B, 1, O), lambda i: (0, 0, 0)),
            pl.BlockSpec((1, O), lambda i: (0, 0)),
            pl.BlockSpec((1, O), lambda i: (0, 0)),
        ],
        out_specs=pl.BlockSpec((1, N, O), lambda i: (i, 0, 0)),
        out_shape=jax.ShapeDtypeStruct((B, N, O), f32),
    )(ymax, ymin, s1, s2, g.reshape(1, O), b.reshape(1, O))


def _edge_layer_fused(xt, W, g, b):
    B, N, C = xt.shape
    O = W.shape[0]
    f32 = jnp.float32
    wT = jnp.transpose(W)

    ymax, ymin, s1, s2 = pl.pallas_call(
        _layer_a_body,
        grid=(B,),
        in_specs=[
            pl.BlockSpec((1, N, C), lambda i: (i, 0, 0)),
            pl.BlockSpec((2 * C, O), lambda i: (0, 0)),
        ],
        out_specs=[
            pl.BlockSpec((1, N, O), lambda i: (i, 0, 0)),
            pl.BlockSpec((1, N, O), lambda i: (i, 0, 0)),
            pl.BlockSpec((1, 1, O), lambda i: (i, 0, 0)),
            pl.BlockSpec((1, 1, O), lambda i: (i, 0, 0)),
        ],
        out_shape=[
            jax.ShapeDtypeStruct((B, N, O), f32),
            jax.ShapeDtypeStruct((B, N, O), f32),
            jax.ShapeDtypeStruct((B, 1, O), f32),
            jax.ShapeDtypeStruct((B, 1, O), f32),
        ],
        scratch_shapes=[
            pltpu.VMEM((N, N), f32),
            pltpu.VMEM((N, O), f32),
            pltpu.VMEM((N, O), f32),
            pltpu.VMEM((N, O), f32),
            pltpu.VMEM((N, O), f32),
        ],
    )(xt, wT)
    return _bn_finalize(ymax, ymin, s1, s2, g, b)


def _topk_body(xt_ref, idx_ref, d_ref):
    xt = xt_ref[0]
    n = xt.shape[0]
    b = pl.program_id(0)
    d, c_i = _pairwise_dist(xt, n)
    d_ref[...] = d
    for k in range(_K):
        d = d_ref[...]
        m = jnp.max(d, axis=1, keepdims=True)
        masked = jnp.where(d == m, c_i, n)
        amin = jnp.min(masked, axis=1, keepdims=True)
        sel = masked == amin
        d_ref[...] = jnp.where(sel, _NEG, d)
        idx_ref[0, :, k:k + 1] = amin + b * n


def _topk_indices(xt):
    B, N, C = xt.shape
    return pl.pallas_call(
        _topk_body,
        grid=(B,),
        in_specs=[pl.BlockSpec((1, N, C), lambda i: (i, 0, 0))],
        out_specs=pl.BlockSpec((1, N, _K), lambda i: (i, 0, 0)),
        out_shape=jax.ShapeDtypeStruct((B, N, _K), jnp.int32),
        scratch_shapes=[pltpu.VMEM((N, N), jnp.float32)],
    )(xt)


def _sc_gather(table, idx_flat):
    V, C = table.shape
    R = idx_flat.shape[0]
    info = plsc.get_sparse_core_info()
    nw = info.num_cores * info.num_subcores
    per_w = R // nw
    ch = 128
    n_ch = per_w // ch
    mesh = plsc.VectorSubcoreMesh(core_axis_name="c", subcore_axis_name="s")

    @functools.partial(
        pl.kernel,
        mesh=mesh,
        out_type=jax.ShapeDtypeStruct((R, C), jnp.float32),
        scratch_types=[
            pltpu.VMEM((ch,), jnp.int32),
            pltpu.VMEM((ch, C), jnp.float32),
            pltpu.SemaphoreType.DMA,
        ],
    )
    def gather_kernel(table_hbm, idx_hbm, out_hbm, idx_v, rows_v, sem):
        wid = lax.axis_index("s") * info.num_cores + lax.axis_index("c")
        base = wid * per_w

        def body(j, carry):
            off = base + j * ch
            pltpu.sync_copy(idx_hbm.at[pl.ds(off, ch)], idx_v)
            pltpu.async_copy(table_hbm.at[idx_v], rows_v, sem).wait()
            pltpu.sync_copy(rows_v, out_hbm.at[pl.ds(off, ch)])
            return carry

        lax.fori_loop(0, n_ch, body, 0)

    return gather_kernel(table, idx_flat)


def _conv_pool_body(feat_ref, xt_ref, wT_ref,
                    ymax_ref, ymin_ref, s1_ref, s2_ref,
                    mx_s, mn_s, s_s, q_s):
    k = pl.program_id(1)
    f32 = jnp.float32
    xi = xt_ref[0]
    c = xi.shape[1]
    xg = feat_ref[0, 0][:, :c]
    e = jnp.concatenate([xg - xi, xi], axis=1)
    y = jax.lax.dot_general(e, wT_ref[...], (((1,), (0,)), ((), ())),
                            preferred_element_type=f32)

    @pl.when(k == 0)
    def _():
        mx_s[...] = jnp.maximum(jnp.full(mx_s.shape, _NEG, f32), y)
        mn_s[...] = jnp.minimum(jnp.full(mn_s.shape, -_NEG, f32), y)
        s_s[...] = y
        q_s[...] = y * y

    @pl.when(k > 0)
    def _():
        mx_s[...] = jnp.maximum(mx_s[...], y)
        mn_s[...] = jnp.minimum(mn_s[...], y)
        s_s[...] = s_s[...] + y
        q_s[...] = q_s[...] + y * y

    @pl.when(k == _K - 1)
    def _():
        ymax_ref[0] = mx_s[...]
        ymin_ref[0] = mn_s[...]
        s1_ref[0] = jnp.sum(s_s[...], axis=0, keepdims=True)
        s2_ref[0] = jnp.sum(q_s[...], axis=0, keepdims=True)


def _edge_layer_sc(xt, W, g, b):
    B, N, C = xt.shape
    O = W.shape[0]
    f32 = jnp.float32
    wT = jnp.transpose(W)

    idx = _topk_indices(xt)
    idx_flat = jnp.transpose(idx, (0, 2, 1)).reshape(B * _K * N)
    Cp = max(C, 128)
    table = xt.reshape(B * N, C)
    if Cp != C:
        table = jnp.concatenate(
            [table, jnp.zeros((B * N, Cp - C), jnp.float32)], axis=1)
    feat = _sc_gather(table, idx_flat)
    feat = feat.reshape(B, _K, N, Cp)

    ymax, ymin, s1, s2 = pl.pallas_call(
        _conv_pool_body,
        grid=(B, _K),
        in_specs=[
            pl.BlockSpec((1, 1, N, Cp), lambda i, k: (i, k, 0, 0)),
            pl.BlockSpec((1, N, C), lambda i, k: (i, 0, 0)),
            pl.BlockSpec((2 * C, O), lambda i, k: (0, 0)),
        ],
        out_specs=[
            pl.BlockSpec((1, N, O), lambda i, k: (i, 0, 0)),
            pl.BlockSpec((1, N, O), lambda i, k: (i, 0, 0)),
            pl.BlockSpec((1, 1, O), lambda i, k: (i, 0, 0)),
            pl.BlockSpec((1, 1, O), lambda i, k: (i, 0, 0)),
        ],
        out_shape=[
            jax.ShapeDtypeStruct((B, N, O), f32),
            jax.ShapeDtypeStruct((B, N, O), f32),
            jax.ShapeDtypeStruct((B, 1, O), f32),
            jax.ShapeDtypeStruct((B, 1, O), f32),
        ],
        scratch_shapes=[
            pltpu.VMEM((N, O), f32),
            pltpu.VMEM((N, O), f32),
            pltpu.VMEM((N, O), f32),
            pltpu.VMEM((N, O), f32),
        ],
    )(feat, xt, wT)
    return _bn_finalize(ymax, ymin, s1, s2, g, b)


def kernel(x, W1, W2, W3, W4, g1, b1, g2, b2, g3, b3, g4, b4):
    h = _edge_layer_fused(x, W1, g1, b1)
    h = _edge_layer_sc(h, W2, g2, b2)
    h = _edge_layer_sc(h, W3, g3, b3)
    h = _edge_layer_sc(h, W4, g4, b4)
    return h
